# baseline (speedup 1.0000x reference)
import numpy as np
import jax
import jax.numpy as jnp
from jax import lax
from jax.experimental import pallas as pl
from jax.experimental.pallas import tpu as pltpu
from jax.experimental.pallas import tpu_sc as plsc

_N_TOKENS = 16384
_N_EXPERTS = 64
_NC = 2
_NS = 16
_NW = _NC * _NS
_L = 16
_RW = _N_TOKENS // _NW
_CHUNKS = _RW // _L
_LN2 = float(np.log(2.0))


def _vlog(s):
    bits = lax.bitcast_convert_type(s, jnp.int32)
    k = (bits >> 23) - 127
    f = lax.bitcast_convert_type((bits & 0x007FFFFF) | 0x3F800000, jnp.float32)
    t = (f - 1.0) / (f + 1.0)
    t2 = t * t
    lf = 2.0 * t * (1.0 + t2 * (1.0 / 3.0 + t2 * (0.2 + t2 * (1.0 / 7.0 + t2 * (1.0 / 9.0)))))
    return k.astype(jnp.float32) * _LN2 + lf


def _sc_body(x_hbm, cnt_hbm, prob_hbm, z_hbm, xbuf, rcpbuf, pout, cacc, zbuf):
    wid = lax.axis_index("s") * _NC + lax.axis_index("c")
    pltpu.sync_copy(x_hbm.at[:, pl.ds(wid * _RW, _RW)], xbuf)

    lane = lax.iota(jnp.int32, _L)
    zero16 = jnp.zeros((_L,), jnp.float32)
    ones16 = jnp.ones((_L,), jnp.float32)
    for e in range(_N_EXPERTS):
        cacc[e, :] = zero16

    def chunk_body(c, zacc):
        t0 = c * _L
        s = None
        best = None
        for g in range(8):
            xs = [(xbuf[8 * g + j, pl.ds(t0, _L)], 8 * g + j) for j in range(8)]
            nodes = [(x, jnp.full((_L,), e, jnp.int32)) for x, e in xs]
            while len(nodes) > 1:
                nxt = []
                for a in range(0, len(nodes), 2):
                    (va, ia), (vb, ib) = nodes[a], nodes[a + 1]
                    upd = vb > va
                    nxt.append((jnp.where(upd, vb, va), jnp.where(upd, ib, ia)))
                nodes = nxt
            gv, gi = nodes[0]
            if best is None:
                best, besti = gv, gi
            else:
                upd = gv > best
                best = jnp.where(upd, gv, best)
                besti = jnp.where(upd, gi, besti)
            ps = [jnp.exp(x) for x, _ in xs]
            gs = ((ps[0] + ps[1]) + (ps[2] + ps[3])) + ((ps[4] + ps[5]) + (ps[6] + ps[7]))
            s = gs if s is None else s + gs
        plsc.addupdate_scatter(cacc, [besti, lane], ones16)
        logz = _vlog(s)
        rcpbuf[pl.ds(t0, _L)] = 1.0 / s
        return zacc + logz * logz

    zacc = lax.fori_loop(0, _CHUNKS, chunk_body, zero16)
    zbuf[:] = zacc

    for eblk in range(8):
        def blk_body(c, accs, eblk=eblk):
            t0 = c * _L
            rcp = rcpbuf[pl.ds(t0, _L)]
            return tuple(
                acc + rcp * jnp.exp(xbuf[8 * eblk + j, pl.ds(t0, _L)])
                for j, acc in enumerate(accs)
            )
        accs = lax.fori_loop(0, _CHUNKS, blk_body, (zero16,) * 8)
        for j in range(8):
            pout[8 * eblk + j, :] = accs[j]

    pltpu.sync_copy(cacc, cnt_hbm.at[wid])
    pltpu.sync_copy(pout, prob_hbm.at[wid])
    pltpu.sync_copy(zbuf, z_hbm.at[wid])


_sc_main = pl.kernel(
    _sc_body,
    out_type=(
        jax.ShapeDtypeStruct((_NW, _N_EXPERTS, _L), jnp.float32),
        jax.ShapeDtypeStruct((_NW, _N_EXPERTS, _L), jnp.float32),
        jax.ShapeDtypeStruct((_NW, _L), jnp.float32),
    ),
    mesh=plsc.VectorSubcoreMesh(core_axis_name="c", subcore_axis_name="s"),
    compiler_params=pltpu.CompilerParams(
        needs_layout_passes=False, use_tc_tiling_on_sc=True
    ),
    scratch_types=[
        pltpu.VMEM((_N_EXPERTS, _RW), jnp.float32),
        pltpu.VMEM((_RW,), jnp.float32),
        pltpu.VMEM((_N_EXPERTS, _L), jnp.float32),
        pltpu.VMEM((_N_EXPERTS, _L), jnp.float32),
        pltpu.VMEM((_L,), jnp.float32),
    ],
)


def _fin_body(cnt_ref, prob_ref, z_ref, out_ref):
    cnt = jnp.sum(cnt_ref[...], axis=0)
    prob = jnp.sum(prob_ref[...], axis=0)
    csum = jnp.sum(cnt, axis=1)
    psum = jnp.sum(prob, axis=1)
    z_loss = jnp.sum(z_ref[...]) / _N_TOKENS
    aux = jnp.sum((csum / _N_TOKENS) * (psum / _N_TOKENS)) * _N_EXPERTS
    out_ref[...] = jnp.reshape(0.001 * z_loss + 0.001 * aux, (1, 1))


_finisher = pl.pallas_call(
    _fin_body,
    out_shape=jax.ShapeDtypeStruct((1, 1), jnp.float32),
)


def kernel(router_logits, attention_mask):
    del attention_mask
    cnt, prob, z = _sc_main(router_logits.T)
    return _finisher(cnt, prob, z)[0, 0]


# speedup vs baseline: 1.0019x; 1.0019x over previous
import numpy as np
import jax
import jax.numpy as jnp
from jax import lax
from jax.experimental import pallas as pl
from jax.experimental.pallas import tpu as pltpu
from jax.experimental.pallas import tpu_sc as plsc

_N_TOKENS = 16384
_N_EXPERTS = 64
_NC = 2
_NS = 16
_NW = _NC * _NS
_L = 16
_RW = _N_TOKENS // _NW
_CHUNKS = _RW // _L
_LN2 = float(np.log(2.0))


def _vlog(s):
    bits = lax.bitcast_convert_type(s, jnp.int32)
    k = (bits >> 23) - 127
    f = lax.bitcast_convert_type((bits & 0x007FFFFF) | 0x3F800000, jnp.float32)
    t = (f - 1.0) / (f + 1.0)
    t2 = t * t
    lf = 2.0 * t * (1.0 + t2 * (1.0 / 3.0 + t2 * (0.2 + t2 * (1.0 / 7.0 + t2 * (1.0 / 9.0)))))
    return k.astype(jnp.float32) * _LN2 + lf


_SLABS = 4
_ST = _RW // _SLABS
_SCHUNKS = _ST // _L


def _sc_body(x_hbm, cnt_hbm, prob_hbm, z_hbm, xa, xb, ptmp, rcpbuf, pout, cacc,
             zbuf, sema, semb):
    wid = lax.axis_index("s") * _NC + lax.axis_index("c")
    t_base = wid * _RW

    lane = lax.iota(jnp.int32, _L)
    zero16 = jnp.zeros((_L,), jnp.float32)
    ones16 = jnp.ones((_L,), jnp.float32)
    for e in range(_N_EXPERTS):
        cacc[e, :] = zero16

    bufs = [xa, xb]
    sems = [sema, semb]
    pend = pltpu.async_copy(x_hbm.at[:, pl.ds(t_base, _ST)], xa, sema)

    zacc = zero16
    for sl in range(_SLABS):
        pend.wait()
        if sl + 1 < _SLABS:
            pend = pltpu.async_copy(
                x_hbm.at[:, pl.ds(t_base + (sl + 1) * _ST, _ST)],
                bufs[(sl + 1) % 2], sems[(sl + 1) % 2])
        buf = bufs[sl % 2]

        def chunk_body(c, zacc, buf=buf, sl=sl):
            t0 = c * _L
            tg = sl * _ST + t0
            s = None
            best = None
            for g in range(8):
                xs = [(buf[8 * g + j, pl.ds(t0, _L)], 8 * g + j) for j in range(8)]
                nodes = [(x, jnp.full((_L,), e, jnp.int32)) for x, e in xs]
                while len(nodes) > 1:
                    nxt = []
                    for a in range(0, len(nodes), 2):
                        (va, ia), (vb, ib) = nodes[a], nodes[a + 1]
                        upd = vb > va
                        nxt.append((jnp.where(upd, vb, va), jnp.where(upd, ib, ia)))
                    nodes = nxt
                gv, gi = nodes[0]
                if best is None:
                    best, besti = gv, gi
                else:
                    upd = gv > best
                    best = jnp.where(upd, gv, best)
                    besti = jnp.where(upd, gi, besti)
                ps = [jnp.exp(x) for x, _ in xs]
                for j in range(8):
                    ptmp[8 * g + j, pl.ds(tg, _L)] = ps[j]
                gs = ((ps[0] + ps[1]) + (ps[2] + ps[3])) + ((ps[4] + ps[5]) + (ps[6] + ps[7]))
                s = gs if s is None else s + gs
            plsc.addupdate_scatter(cacc, [besti, lane], ones16)
            logz = _vlog(s)
            rcpbuf[pl.ds(tg, _L)] = 1.0 / s
            return zacc + logz * logz

        zacc = lax.fori_loop(0, _SCHUNKS, chunk_body, zacc)

    zbuf[:] = zacc

    for eblk in range(8):
        def blk_body(c, accs, eblk=eblk):
            t0 = c * _L
            rcp = rcpbuf[pl.ds(t0, _L)]
            return tuple(
                acc + rcp * ptmp[8 * eblk + j, pl.ds(t0, _L)]
                for j, acc in enumerate(accs)
            )
        accs = lax.fori_loop(0, _CHUNKS, blk_body, (zero16,) * 8)
        for j in range(8):
            pout[8 * eblk + j, :] = accs[j]

    pltpu.sync_copy(cacc, cnt_hbm.at[wid])
    pltpu.sync_copy(pout, prob_hbm.at[wid])
    pltpu.sync_copy(zbuf, z_hbm.at[wid])


_sc_main = pl.kernel(
    _sc_body,
    out_type=(
        jax.ShapeDtypeStruct((_NW, _N_EXPERTS, _L), jnp.float32),
        jax.ShapeDtypeStruct((_NW, _N_EXPERTS, _L), jnp.float32),
        jax.ShapeDtypeStruct((_NW, _L), jnp.float32),
    ),
    mesh=plsc.VectorSubcoreMesh(core_axis_name="c", subcore_axis_name="s"),
    compiler_params=pltpu.CompilerParams(
        needs_layout_passes=False, use_tc_tiling_on_sc=True
    ),
    scratch_types=[
        pltpu.VMEM((_N_EXPERTS, _ST), jnp.float32),
        pltpu.VMEM((_N_EXPERTS, _ST), jnp.float32),
        pltpu.VMEM((_N_EXPERTS, _RW), jnp.float32),
        pltpu.VMEM((_RW,), jnp.float32),
        pltpu.VMEM((_N_EXPERTS, _L), jnp.float32),
        pltpu.VMEM((_N_EXPERTS, _L), jnp.float32),
        pltpu.VMEM((_L,), jnp.float32),
        pltpu.SemaphoreType.DMA,
        pltpu.SemaphoreType.DMA,
    ],
)


def _fin_body(cnt_ref, prob_ref, z_ref, out_ref):
    cnt = jnp.sum(cnt_ref[...], axis=0)
    prob = jnp.sum(prob_ref[...], axis=0)
    csum = jnp.sum(cnt, axis=1)
    psum = jnp.sum(prob, axis=1)
    z_loss = jnp.sum(z_ref[...]) / _N_TOKENS
    aux = jnp.sum((csum / _N_TOKENS) * (psum / _N_TOKENS)) * _N_EXPERTS
    out_ref[...] = jnp.reshape(0.001 * z_loss + 0.001 * aux, (1, 1))


_finisher = pl.pallas_call(
    _fin_body,
    out_shape=jax.ShapeDtypeStruct((1, 1), jnp.float32),
)


def kernel(router_logits, attention_mask):
    del attention_mask
    cnt, prob, z = _sc_main(router_logits.T)
    return _finisher(cnt, prob, z)[0, 0]


# speedup vs baseline: 1.1216x; 1.1195x over previous
import numpy as np
import jax
import jax.numpy as jnp
from jax import lax
from jax.experimental import pallas as pl
from jax.experimental.pallas import tpu as pltpu
from jax.experimental.pallas import tpu_sc as plsc

_N_TOKENS = 16384
_N_EXPERTS = 64
_NC = 2
_NS = 16
_NW = _NC * _NS
_L = 16

_N_SC = 8192
_N_TC = _N_TOKENS - _N_SC
_RW = _N_SC // _NW
_CHUNKS = _RW // _L
_ST = 128
_SLABS = _RW // _ST
_SCHUNKS = _ST // _L
_TB = 512
_LN2 = float(np.log(2.0))


def _vlog(s):
    bits = lax.bitcast_convert_type(s, jnp.int32)
    k = (bits >> 23) - 127
    f = lax.bitcast_convert_type((bits & 0x007FFFFF) | 0x3F800000, jnp.float32)
    t = (f - 1.0) / (f + 1.0)
    t2 = t * t
    lf = 2.0 * t * (1.0 + t2 * (1.0 / 3.0 + t2 * (0.2 + t2 * (1.0 / 7.0 + t2 * (1.0 / 9.0)))))
    return k.astype(jnp.float32) * _LN2 + lf


def _sc_body(x_hbm, cnt_hbm, prob_hbm, z_hbm, xa, xb, ptmp, rcpbuf, pout, cacc,
             zbuf, sema, semb):
    wid = lax.axis_index("s") * _NC + lax.axis_index("c")
    t_base = wid * _RW

    lane = lax.iota(jnp.int32, _L)
    zero16 = jnp.zeros((_L,), jnp.float32)
    ones16 = jnp.ones((_L,), jnp.float32)
    for e in range(_N_EXPERTS):
        cacc[e, :] = zero16

    bufs = [xa, xb]
    sems = [sema, semb]
    pend = pltpu.async_copy(x_hbm.at[:, pl.ds(t_base, _ST)], xa, sema)

    zacc = zero16
    for sl in range(_SLABS):
        pend.wait()
        if sl + 1 < _SLABS:
            pend = pltpu.async_copy(
                x_hbm.at[:, pl.ds(t_base + (sl + 1) * _ST, _ST)],
                bufs[(sl + 1) % 2], sems[(sl + 1) % 2])
        buf = bufs[sl % 2]

        def chunk_body(c, zacc, buf=buf, sl=sl):
            t0 = c * _L
            tg = sl * _ST + t0
            s = None
            best = None
            for g in range(8):
                xs = [(buf[8 * g + j, pl.ds(t0, _L)], 8 * g + j) for j in range(8)]
                nodes = [(x, jnp.full((_L,), e, jnp.int32)) for x, e in xs]
                while len(nodes) > 1:
                    nxt = []
                    for a in range(0, len(nodes), 2):
                        (va, ia), (vb, ib) = nodes[a], nodes[a + 1]
                        upd = vb > va
                        nxt.append((jnp.where(upd, vb, va), jnp.where(upd, ib, ia)))
                    nodes = nxt
                gv, gi = nodes[0]
                if best is None:
                    best, besti = gv, gi
                else:
                    upd = gv > best
                    best = jnp.where(upd, gv, best)
                    besti = jnp.where(upd, gi, besti)
                ps = [jnp.exp(x) for x, _ in xs]
                for j in range(8):
                    ptmp[8 * g + j, pl.ds(tg, _L)] = ps[j]
                gs = ((ps[0] + ps[1]) + (ps[2] + ps[3])) + ((ps[4] + ps[5]) + (ps[6] + ps[7]))
                s = gs if s is None else s + gs
            plsc.addupdate_scatter(cacc, [besti, lane], ones16)
            logz = _vlog(s)
            rcpbuf[pl.ds(tg, _L)] = 1.0 / s
            return zacc + logz * logz

        zacc = lax.fori_loop(0, _SCHUNKS, chunk_body, zacc)

    zbuf[:] = zacc

    for eblk in range(8):
        def blk_body(c, accs, eblk=eblk):
            t0 = c * _L
            rcp = rcpbuf[pl.ds(t0, _L)]
            return tuple(
                acc + rcp * ptmp[8 * eblk + j, pl.ds(t0, _L)]
                for j, acc in enumerate(accs)
            )
        accs = lax.fori_loop(0, _CHUNKS, blk_body, (zero16,) * 8)
        for j in range(8):
            pout[8 * eblk + j, :] = accs[j]

    pltpu.sync_copy(cacc, cnt_hbm.at[wid])
    pltpu.sync_copy(pout, prob_hbm.at[wid])
    pltpu.sync_copy(zbuf, z_hbm.at[wid])


_sc_main = pl.kernel(
    _sc_body,
    out_type=(
        jax.ShapeDtypeStruct((_NW, _N_EXPERTS, _L), jnp.float32),
        jax.ShapeDtypeStruct((_NW, _N_EXPERTS, _L), jnp.float32),
        jax.ShapeDtypeStruct((_NW, _L), jnp.float32),
    ),
    mesh=plsc.VectorSubcoreMesh(core_axis_name="c", subcore_axis_name="s"),
    compiler_params=pltpu.CompilerParams(
        needs_layout_passes=False, use_tc_tiling_on_sc=True
    ),
    scratch_types=[
        pltpu.VMEM((_N_EXPERTS, _ST), jnp.float32),
        pltpu.VMEM((_N_EXPERTS, _ST), jnp.float32),
        pltpu.VMEM((_N_EXPERTS, _RW), jnp.float32),
        pltpu.VMEM((_RW,), jnp.float32),
        pltpu.VMEM((_N_EXPERTS, _L), jnp.float32),
        pltpu.VMEM((_N_EXPERTS, _L), jnp.float32),
        pltpu.VMEM((_L,), jnp.float32),
        pltpu.SemaphoreType.DMA,
        pltpu.SemaphoreType.DMA,
    ],
)


def _tc_body(x_ref, cnt_ref, prob_ref, z_ref):
    i = pl.program_id(0)
    x = x_ref[...]
    m = jnp.max(x, axis=0, keepdims=True)
    e = jnp.exp(x - m)
    s = jnp.sum(e, axis=0, keepdims=True)
    logz = m + jnp.log(s)
    p = e / s
    iot = lax.broadcasted_iota(jnp.int32, (_N_EXPERTS, _TB), 0)
    cand = jnp.where(x == m, iot, _N_EXPERTS)
    am = jnp.min(cand, axis=0, keepdims=True)
    onehot = (iot == am).astype(jnp.float32)

    @pl.when(i == 0)
    def _init():
        cnt_ref[...] = jnp.zeros_like(cnt_ref)
        prob_ref[...] = jnp.zeros_like(prob_ref)
        z_ref[...] = jnp.zeros_like(z_ref)

    cnt_ref[...] += onehot
    prob_ref[...] += p
    z_ref[...] += logz * logz


_tc_share = pl.pallas_call(
    _tc_body,
    grid=(_N_TC // _TB,),
    in_specs=[pl.BlockSpec((_N_EXPERTS, _TB), lambda i: (0, (_N_SC // _TB) + i))],
    out_specs=(
        pl.BlockSpec((_N_EXPERTS, _TB), lambda i: (0, 0)),
        pl.BlockSpec((_N_EXPERTS, _TB), lambda i: (0, 0)),
        pl.BlockSpec((1, _TB), lambda i: (0, 0)),
    ),
    out_shape=(
        jax.ShapeDtypeStruct((_N_EXPERTS, _TB), jnp.float32),
        jax.ShapeDtypeStruct((_N_EXPERTS, _TB), jnp.float32),
        jax.ShapeDtypeStruct((1, _TB), jnp.float32),
    ),
)


def _fin_body(cnt_sc_ref, prob_sc_ref, z_sc_ref, cnt_tc_ref, prob_tc_ref,
              z_tc_ref, out_ref):
    csum = (jnp.sum(jnp.sum(cnt_sc_ref[...], axis=0), axis=1)
            + jnp.sum(cnt_tc_ref[...], axis=1))
    psum = (jnp.sum(jnp.sum(prob_sc_ref[...], axis=0), axis=1)
            + jnp.sum(prob_tc_ref[...], axis=1))
    z_loss = (jnp.sum(z_sc_ref[...]) + jnp.sum(z_tc_ref[...])) / _N_TOKENS
    aux = jnp.sum((csum / _N_TOKENS) * (psum / _N_TOKENS)) * _N_EXPERTS
    out_ref[...] = jnp.reshape(0.001 * z_loss + 0.001 * aux, (1, 1))


_finisher = pl.pallas_call(
    _fin_body,
    out_shape=jax.ShapeDtypeStruct((1, 1), jnp.float32),
)


def kernel(router_logits, attention_mask):
    del attention_mask
    xt = router_logits.T
    cnt_sc, prob_sc, z_sc = _sc_main(xt)
    cnt_tc, prob_tc, z_tc = _tc_share(xt)
    return _finisher(cnt_sc, prob_sc, z_sc, cnt_tc, prob_tc, z_tc)[0, 0]
